# baseline (speedup 1.0000x reference)
import functools

import jax
import jax.numpy as jnp
from jax import lax
from jax.experimental import pallas as pl
from jax.experimental.pallas import tpu as pltpu
from jax.experimental.pallas import tpu_sc as plsc

_NW = 32


def _sc_kernel(B, V, logits_hbm, probs_hbm, dummy):
    RPW = B // _NW
    wid = lax.axis_index("s") * 2 + lax.axis_index("c")
    base = wid * RPW
    pltpu.sync_copy(logits_hbm.at[pl.ds(base, RPW), :],
                    probs_hbm.at[pl.ds(base, RPW), :])


def kernel(logits, actions):
    B, V = logits.shape
    mesh = plsc.VectorSubcoreMesh(core_axis_name="c", subcore_axis_name="s")
    sc = functools.partial(
        pl.kernel,
        out_type=[jax.ShapeDtypeStruct((B, V), jnp.float32)],
        mesh=mesh,
        compiler_params=pltpu.CompilerParams(
            needs_layout_passes=False, use_tc_tiling_on_sc=False),
        scratch_types=[pltpu.VMEM((16,), jnp.float32)],
    )(functools.partial(_sc_kernel, B, V))
    (p,) = sc(logits)
    return p


# speedup vs baseline: 8.1690x; 8.1690x over previous
import functools

import jax
import jax.numpy as jnp
from jax import lax
from jax.experimental import pallas as pl
from jax.experimental.pallas import tpu as pltpu
from jax.experimental.pallas import tpu_sc as plsc


_NC, _NS, _L = 2, 16, 16
_NW = _NC * _NS
_UN = 5
_SPLIT = 768
_TC_R = 16


def _tc_kernel(logits_ref, act_ref, lp_ref, mode_ref, probs_ref):
    x = logits_ref[...]
    a = act_ref[...]
    m = jnp.max(x, axis=-1, keepdims=True)
    e = jnp.exp(x - m)
    s = jnp.sum(e, axis=-1, keepdims=True)
    probs_ref[...] = e * (100.0 / s)

    cols = jax.lax.broadcasted_iota(jnp.int32, x.shape, 1)
    big = jnp.int32(x.shape[-1])
    mode_ref[...] = jnp.min(jnp.where(x == m, cols, big), axis=-1,
                            keepdims=True)
    g = jnp.max(jnp.where(cols == a, x, -jnp.inf), axis=-1, keepdims=True)
    lp_ref[...] = g - m - jnp.log(s)


def _sc_kernel(B, V, logits_hbm, actions_hbm, g_hbm, m_hbm, s_hbm,
               mode_hbm, probs_hbm, xbuf, abuf, gbuf, mbuf, sbuf, modebuf):
    RPW = (B - _SPLIT) // _NW
    NCH = V // _L
    wid = lax.axis_index("s") * _NC + lax.axis_index("c")
    out_base = wid * RPW
    base = _SPLIT + out_base
    lane = lax.iota(jnp.int32, _L)
    ninf = jnp.full((_L,), -jnp.inf, jnp.float32)
    zero_i = jnp.zeros((_L,), jnp.int32)
    zero_f = jnp.zeros((_L,), jnp.float32)

    pltpu.sync_copy(actions_hbm.at[pl.ds(base, RPW)],
                    abuf.at[pl.ds(0, RPW)])
    avec = abuf[pl.ds(0, _L)]
    g_acc, m_acc, s_acc = zero_f, zero_f, zero_f
    i_acc = zero_i
    for t in range(RPW):
        row = base + t
        pltpu.sync_copy(logits_hbm.at[row], xbuf.at[pl.ds(0, V)])

        def p1(it, acc):
            vms, vis = acc[:_UN], acc[_UN:]
            new_vms, new_vis = [], []
            for k in range(_UN):
                j = it * _UN + k
                v = xbuf[pl.ds(j * _L, _L)]
                gt = v > vms[k]
                new_vis.append(jnp.where(gt, j, vis[k]))
                new_vms.append(jnp.maximum(v, vms[k]))
            return tuple(new_vms) + tuple(new_vis)

        acc = lax.fori_loop(0, NCH // _UN, p1,
                            (ninf,) * _UN + (zero_i,) * _UN)
        vms, vis = acc[:_UN], acc[_UN:]
        m = jnp.max(vms[0])
        for k in range(1, _UN):
            m = jnp.maximum(m, jnp.max(vms[k]))
        idx = jnp.int32(V)
        for k in range(_UN):
            cand = jnp.where(vms[k] == m, vis[k] * _L + lane, jnp.int32(V))
            idx = jnp.minimum(idx, jnp.min(cand))

        a = avec[t]
        g = xbuf[pl.ds(a, _L)][0]

        mv = jnp.full((_L,), m, jnp.float32)

        def p2(it, ss):
            out = []
            for k in range(_UN):
                j = it * _UN + k
                e = jnp.exp(xbuf[pl.ds(j * _L, _L)] - mv)
                xbuf[pl.ds(j * _L, _L)] = e
                out.append(ss[k] + e)
            return tuple(out)

        ss = lax.fori_loop(0, NCH // _UN, p2, (zero_f,) * _UN)
        s = jnp.sum(ss[0])
        for k in range(1, _UN):
            s = s + jnp.sum(ss[k])

        sv = jnp.full((_L,), s, jnp.float32)
        rv = jnp.full((_L,), 100.0, jnp.float32) / sv

        def p3(it, c):
            for k in range(_UN):
                j = it * _UN + k
                xbuf[pl.ds(j * _L, _L)] = xbuf[pl.ds(j * _L, _L)] * rv
            return c

        lax.fori_loop(0, NCH // _UN, p3, 0)
        pltpu.sync_copy(xbuf.at[pl.ds(0, V)],
                        probs_hbm.at[out_base + t])

        here = lane == t
        g_acc = jnp.where(here, jnp.full((_L,), g), g_acc)
        m_acc = jnp.where(here, mv, m_acc)
        s_acc = jnp.where(here, jnp.full((_L,), s), s_acc)
        i_acc = jnp.where(here, jnp.full((_L,), idx), i_acc)

    gbuf[pl.ds(0, _L)] = g_acc
    mbuf[pl.ds(0, _L)] = m_acc
    sbuf[pl.ds(0, _L)] = s_acc
    modebuf[pl.ds(0, _L)] = i_acc

    pltpu.sync_copy(gbuf.at[pl.ds(0, RPW)], g_hbm.at[pl.ds(out_base, RPW)])
    pltpu.sync_copy(mbuf.at[pl.ds(0, RPW)], m_hbm.at[pl.ds(out_base, RPW)])
    pltpu.sync_copy(sbuf.at[pl.ds(0, RPW)], s_hbm.at[pl.ds(out_base, RPW)])
    pltpu.sync_copy(modebuf.at[pl.ds(0, RPW)],
                    mode_hbm.at[pl.ds(out_base, RPW)])


def _lp_kernel(g_ref, m_ref, s_ref, lp_ref):
    lp_ref[...] = g_ref[...] - m_ref[...] - jnp.log(s_ref[...])


def kernel(logits, actions):
    B, V = logits.shape
    SB = B - _SPLIT
    RPW = SB // _NW

    mesh = plsc.VectorSubcoreMesh(core_axis_name="c", subcore_axis_name="s")
    sc = functools.partial(
        pl.kernel,
        out_type=[
            jax.ShapeDtypeStruct((SB,), jnp.float32),
            jax.ShapeDtypeStruct((SB,), jnp.float32),
            jax.ShapeDtypeStruct((SB,), jnp.float32),
            jax.ShapeDtypeStruct((SB,), jnp.int32),
            jax.ShapeDtypeStruct((SB, V), jnp.float32),
        ],
        mesh=mesh,
        compiler_params=pltpu.CompilerParams(
            needs_layout_passes=False, use_tc_tiling_on_sc=False),
        scratch_types=[
            pltpu.VMEM((V + _L,), jnp.float32),
            pltpu.VMEM((_L,), jnp.int32),
            pltpu.VMEM((_L,), jnp.float32),
            pltpu.VMEM((_L,), jnp.float32),
            pltpu.VMEM((_L,), jnp.float32),
            pltpu.VMEM((_L,), jnp.int32),
        ],
    )(functools.partial(_sc_kernel, B, V))
    g_sc, m_sc, s_sc, mode_sc, probs_sc = sc(logits, actions.reshape(B))

    grid = (_SPLIT // _TC_R,)
    lp_tc, mode_tc, probs_full = pl.pallas_call(
        _tc_kernel,
        grid=grid,
        in_specs=[
            pl.BlockSpec((_TC_R, V), lambda i: (i, 0)),
            pl.BlockSpec((_TC_R, 1), lambda i: (i, 0)),
        ],
        out_specs=[
            pl.BlockSpec((_TC_R, 1), lambda i: (i, 0)),
            pl.BlockSpec((_TC_R, 1), lambda i: (i, 0)),
            pl.BlockSpec((_TC_R, V), lambda i: (i, 0)),
        ],
        out_shape=[
            jax.ShapeDtypeStruct((_SPLIT, 1), jnp.float32),
            jax.ShapeDtypeStruct((_SPLIT, 1), jnp.int32),
            jax.ShapeDtypeStruct((B, V), jnp.float32),
        ],
    )(logits, actions)

    lp_sc = pl.pallas_call(
        _lp_kernel,
        out_shape=jax.ShapeDtypeStruct((8, SB // 8), jnp.float32),
    )(g_sc.reshape(8, SB // 8), m_sc.reshape(8, SB // 8),
      s_sc.reshape(8, SB // 8))

    new_probs = lax.dynamic_update_slice(probs_full, probs_sc, (_SPLIT, 0))
    lp = jnp.concatenate([lp_tc, lp_sc.reshape(SB, 1)], axis=0)
    mode_idx = jnp.concatenate([mode_tc, mode_sc.reshape(SB, 1)], axis=0)
    return (lp, mode_idx, new_probs)


# speedup vs baseline: 8.9538x; 1.0961x over previous
import functools

import jax
import jax.numpy as jnp
from jax import lax
from jax.experimental import pallas as pl
from jax.experimental.pallas import tpu as pltpu
from jax.experimental.pallas import tpu_sc as plsc


_NC, _NS, _L = 2, 16, 16
_NW = _NC * _NS
_TC_R = 16


def _sc_gather_kernel(B, V, flat_hbm, actions_hbm, g_hbm,
                      abuf, idxbuf, gbuf, sem):
    RPW = B // _NW
    wid = lax.axis_index("s") * _NC + lax.axis_index("c")
    base = wid * RPW
    lane = lax.iota(jnp.int32, _L)

    pltpu.sync_copy(actions_hbm.at[pl.ds(base, RPW)], abuf)
    for g in range(RPW // _L):
        av = abuf[pl.ds(g * _L, _L)]
        rowid = base + g * _L + lane
        idxbuf[pl.ds(g * _L, _L)] = rowid * V + av
    pltpu.async_copy(flat_hbm.at[idxbuf], gbuf, sem).wait()
    pltpu.sync_copy(gbuf, g_hbm.at[pl.ds(base, RPW)])


def _tc_kernel(logits_ref, g_ref, lp_ref, mode_ref, probs_ref):
    x = logits_ref[...]
    m = jnp.max(x, axis=-1, keepdims=True)
    e = jnp.exp(x - m)
    s = jnp.sum(e, axis=-1, keepdims=True)
    probs_ref[...] = e * (100.0 / s)

    cols = jax.lax.broadcasted_iota(jnp.int32, x.shape, 1)
    big = jnp.int32(x.shape[-1])
    mode_ref[...] = jnp.min(jnp.where(x == m, cols, big), axis=-1,
                            keepdims=True)
    lp_ref[...] = g_ref[...] - m - jnp.log(s)


def kernel(logits, actions):
    B, V = logits.shape
    RPW = B // _NW

    mesh = plsc.VectorSubcoreMesh(core_axis_name="c", subcore_axis_name="s")
    sc = functools.partial(
        pl.kernel,
        out_type=jax.ShapeDtypeStruct((B,), jnp.float32),
        mesh=mesh,
        compiler_params=pltpu.CompilerParams(
            needs_layout_passes=False, use_tc_tiling_on_sc=False),
        scratch_types=[
            pltpu.VMEM((RPW,), jnp.int32),
            pltpu.VMEM((RPW,), jnp.int32),
            pltpu.VMEM((RPW,), jnp.float32),
            pltpu.SemaphoreType.DMA,
        ],
    )(functools.partial(_sc_gather_kernel, B, V))
    g = sc(logits.reshape(B * V), actions.reshape(B))

    grid = (B // _TC_R,)
    lp, mode_idx, new_probs = pl.pallas_call(
        _tc_kernel,
        grid=grid,
        in_specs=[
            pl.BlockSpec((_TC_R, V), lambda i: (i, 0)),
            pl.BlockSpec((_TC_R, 1), lambda i: (i, 0)),
        ],
        out_specs=[
            pl.BlockSpec((_TC_R, 1), lambda i: (i, 0)),
            pl.BlockSpec((_TC_R, 1), lambda i: (i, 0)),
            pl.BlockSpec((_TC_R, V), lambda i: (i, 0)),
        ],
        out_shape=[
            jax.ShapeDtypeStruct((B, 1), jnp.float32),
            jax.ShapeDtypeStruct((B, 1), jnp.int32),
            jax.ShapeDtypeStruct((B, V), jnp.float32),
        ],
    )(logits, g.reshape(B, 1))

    return (lp, mode_idx, new_probs)


# speedup vs baseline: 13.9662x; 1.5598x over previous
import jax
import jax.numpy as jnp
from jax.experimental import pallas as pl


_ROWS = 16


def _fused_kernel(logits_ref, act_ref, lp_ref, mode_ref, probs_ref):
    x = logits_ref[...]
    a = act_ref[...]
    m = jnp.max(x, axis=-1, keepdims=True)
    e = jnp.exp(x - m)
    s = jnp.sum(e, axis=-1, keepdims=True)
    probs_ref[...] = e * (100.0 / s)

    cols = jax.lax.broadcasted_iota(jnp.int32, x.shape, 1)
    big = jnp.int32(x.shape[-1])
    mode_ref[...] = jnp.min(jnp.where(e == 1.0, cols, big), axis=-1,
                            keepdims=True)
    g = jnp.max(jnp.where(cols == a, x, -jnp.inf), axis=-1, keepdims=True)
    lp_ref[...] = g - m - jnp.log(s)


def kernel(logits, actions):
    B, V = logits.shape
    R = _ROWS
    grid = (B // R,)
    lp, mode_idx, new_probs = pl.pallas_call(
        _fused_kernel,
        grid=grid,
        in_specs=[
            pl.BlockSpec((R, V), lambda i: (i, 0)),
            pl.BlockSpec((R, 1), lambda i: (i, 0)),
        ],
        out_specs=[
            pl.BlockSpec((R, 1), lambda i: (i, 0)),
            pl.BlockSpec((R, 1), lambda i: (i, 0)),
            pl.BlockSpec((R, V), lambda i: (i, 0)),
        ],
        out_shape=[
            jax.ShapeDtypeStruct((B, 1), jnp.float32),
            jax.ShapeDtypeStruct((B, 1), jnp.int32),
            jax.ShapeDtypeStruct((B, V), jnp.float32),
        ],
    )(logits, actions)
    return (lp, mode_idx, new_probs)


# speedup vs baseline: 14.0649x; 1.0071x over previous
import jax
import jax.numpy as jnp
from jax.experimental import pallas as pl


_ROWS = 16


def _fused_kernel(logits_ref, act_ref, lp_ref, mode_ref, probs_ref):
    x = logits_ref[...]
    a = act_ref[...]
    m = jnp.max(x, axis=-1, keepdims=True)
    e = jnp.exp(x - m)
    s = jnp.sum(e, axis=-1, keepdims=True)
    probs_ref[...] = e * (100.0 / s)

    cols = jax.lax.broadcasted_iota(jnp.int32, x.shape, 1)
    big = jnp.int32(x.shape[-1])
    mode_ref[...] = jnp.min(jnp.where(x == m, cols, big), axis=-1,
                            keepdims=True)
    g = jnp.max(jnp.where(cols == a, x, -jnp.inf), axis=-1, keepdims=True)
    lp_ref[...] = g - m - jnp.log(s)


def kernel(logits, actions):
    B, V = logits.shape
    R = _ROWS
    grid = (B // R,)
    lp, mode_idx, new_probs = pl.pallas_call(
        _fused_kernel,
        grid=grid,
        in_specs=[
            pl.BlockSpec((R, V), lambda i: (i, 0)),
            pl.BlockSpec((R, 1), lambda i: (i, 0)),
        ],
        out_specs=[
            pl.BlockSpec((R, 1), lambda i: (i, 0)),
            pl.BlockSpec((R, 1), lambda i: (i, 0)),
            pl.BlockSpec((R, V), lambda i: (i, 0)),
        ],
        out_shape=[
            jax.ShapeDtypeStruct((B, 1), jnp.float32),
            jax.ShapeDtypeStruct((B, 1), jnp.int32),
            jax.ShapeDtypeStruct((B, V), jnp.float32),
        ],
    )(logits, actions)
    return (lp, mode_idx, new_probs)
